# baseline (speedup 1.0000x reference)
import functools

import jax
import jax.numpy as jnp
from jax import lax
from jax.experimental import pallas as pl
from jax.experimental.pallas import tpu as pltpu
from jax.experimental.pallas import tpu_sc as plsc

F = 26
B = 4096
VOCAB = 100000
D = 64

NC = 2
NS = 16
SUB = 8
NFP = F // NC
NU = NFP * SUB
W = 6272
W15 = VOCAB - 15 * W
HCAP = B + 64
SLABW = SUB * B
JUNK = 2 * SLABW


def _body(values_hbm, tables_hbm, tail_hbm, out_hbm, vals_v, hv, hb, slab,
          evbuf, oidx, outslab, smem, ssem, scsem, wsem):
  core = lax.axis_index("c")
  t = lax.axis_index("s")
  c0 = t * W
  iota = lax.iota(jnp.int32, 16)

  def scan_feature(fr, width):
    c0v = lax.broadcast_in_dim(c0, (16,), ())
    c1v = lax.broadcast_in_dim(c0 + width, (16,), ())

    def scan_vec(i, carry):
      nh, bvec = carry
      v = vals_v[fr, pl.ds(i * 16, 16)]
      m = (v >= c0v) & (v < c1v)
      plsc.store_compressed(hv.at[pl.ds(nh, 16)], v, mask=m)
      plsc.store_compressed(hb.at[pl.ds(nh, 16)], bvec, mask=m)
      return nh + jnp.sum(m.astype(jnp.int32)), bvec + 16

    nh, _ = lax.fori_loop(0, B // 16, scan_vec, (jnp.int32(0), iota),
                          unroll=False)
    return nh

  def unit_body(u, _):
    buf = lax.rem(u, jnp.int32(2))
    k = lax.div(u, jnp.int32(SUB))
    f = core + NC * k
    s = lax.rem(u, jnp.int32(SUB))
    fr = lax.rem(f, jnp.int32(SUB))

    @pl.when((s == 0) & (lax.rem(k, jnp.int32(4)) == 0))
    def _():
      oct_ = lax.div(f, jnp.int32(SUB))
      pltpu.sync_copy(
          values_hbm.at[pl.ds(pl.multiple_of(oct_ * SUB, SUB), SUB), :],
          vals_v)

    nh = lax.cond(
        s == 0,
        lambda: lax.cond(t == 15,
                         lambda: scan_feature(fr, W15),
                         lambda: scan_feature(fr, W)),
        lambda: smem[0])
    smem[0] = nh

    @pl.when(t < 15)
    def _():
      pltpu.sync_copy(
          tables_hbm.at[f * SUB + s, :, pl.ds(pl.multiple_of(c0, 128), W)],
          slab)

    @pl.when(t == 15)
    def _():
      pltpu.sync_copy(tables_hbm.at[f * SUB + s, :, pl.ds(15 * W, 5888)],
                      slab.at[:, pl.ds(0, 5888)])
      pltpu.sync_copy(tail_hbm.at[f * SUB + s, :, :],
                      slab.at[:, pl.ds(5888, 128)])

    plsc.subcore_barrier()

    nhv = lax.broadcast_in_dim(nh, (16,), ())
    obase = lax.broadcast_in_dim(buf * SLABW, (16,), ())
    ngrp = lax.div(nh + 15, jnp.int32(16))

    def scat_wait1(i, _):
      pltpu.make_async_copy(
          evbuf.at[0],
          outslab.at[oidx.at[0]],
          scsem).wait()
      return 0

    def grp_body(g, carry):
      bvec, = carry
      ring = lax.rem(g, jnp.int32(8))

      hvv = hv[pl.ds(g * 16, 16)]
      hbv = hb[pl.ds(g * 16, 16)]
      ok = bvec < nhv
      shift = jnp.where(hvv >= jnp.int32(VOCAB - 32), jnp.int32(96),
                        jnp.int32(0))
      col = lax.min(lax.max(hvv - c0v16 + shift, zeros16), jnp.int32(W - 1))
      pos0 = ((hbv >> 7) * 1024 + (hbv & 127)) + obase
      for r in range(SUB):
        gv = plsc.load_gather(slab, [jnp.full((16,), r, jnp.int32), col])
        evbuf[ring, pl.ds(r * 16, 16)] = gv
        oidx[ring, pl.ds(r * 16, 16)] = jnp.where(ok, pos0 + r * 128,
                                                  jnp.int32(JUNK))
      pltpu.async_copy(evbuf.at[ring], outslab.at[oidx.at[ring]], scsem).wait()
      return (bvec + 16,)

    c0v16 = lax.broadcast_in_dim(c0, (16,), ())
    zeros16 = jnp.zeros((16,), jnp.int32)
    lax.fori_loop(0, ngrp, grp_body, (iota,), unroll=False)

    plsc.subcore_barrier()

    pltpu.async_copy(
        outslab.at[pl.ds(pl.multiple_of(buf * SLABW + t * 2048, 1024), 2048)],
        out_hbm.at[pl.ds(
            pl.multiple_of(((f * SUB + s) * SUB) * B + t * 2048, 1024), 2048)],
        wsem.at[0]).wait()
    return 0

  lax.fori_loop(0, NU, unit_body, 0, unroll=False)


@jax.jit
def _pooled_lookup(values, tables_3, tail_3):
  mesh = plsc.VectorSubcoreMesh(core_axis_name="c", subcore_axis_name="s")
  fn = pl.kernel(
      _body,
      out_type=jax.ShapeDtypeStruct((F * D * B,), jnp.float32),
      mesh=mesh,
      scratch_types=[
          pltpu.VMEM((SUB, B), jnp.int32),
          pltpu.VMEM((HCAP,), jnp.int32),
          pltpu.VMEM((HCAP,), jnp.int32),
          pltpu.VMEM((SUB, W), jnp.float32),
          pltpu.VMEM((8, 128), jnp.float32),
          pltpu.VMEM((8, 128), jnp.int32),
          pltpu.VMEM_SHARED((2 * SLABW + 128,), jnp.float32),
          pltpu.SMEM((1,), jnp.int32),
          pltpu.SemaphoreType.DMA,
          pltpu.SemaphoreType.DMA,
          pltpu.SemaphoreType.DMA((2,)),
      ],
      compiler_params=pltpu.CompilerParams(
          use_tc_tiling_on_sc=True, needs_layout_passes=False),
  )
  return fn(values, tables_3, tail_3)


def kernel(values, offsets, tables):
  del offsets
  tables_3 = jnp.transpose(tables, (0, 2, 1)).reshape(F * SUB, SUB, VOCAB)
  values_p = jnp.pad(values, ((0, 32 - F), (0, 0)))
  tail_3 = (jnp.transpose(tables[:, VOCAB - 128:, :], (0, 2, 1))
            .reshape(F * SUB, SUB, 128))
  out1 = _pooled_lookup(values_p, tables_3, tail_3)
  out = (out1.reshape(F, SUB, B // 128, SUB, 128)
         .transpose(0, 1, 3, 2, 4)
         .reshape(F, D, B)
         .transpose(2, 0, 1))
  return out


# speedup vs baseline: 1.0954x; 1.0954x over previous
import functools

import jax
import jax.numpy as jnp
from jax import lax
from jax.experimental import pallas as pl
from jax.experimental.pallas import tpu as pltpu
from jax.experimental.pallas import tpu_sc as plsc

F = 26
B = 4096
VOCAB = 100000
D = 64

NC = 2
NS = 16
SUB = 8
NFP = F // NC
NU = NFP * SUB
W = 6272
W15 = VOCAB - 15 * W
HCAP = B + 64
SLABW = SUB * B
JUNK = 2 * SLABW


def _body(values_hbm, tables_hbm, tail_hbm, out_hbm, vals_v, hv, hb, slab,
          evbuf, oidx, outslab, smem, ssem, scsem, wsem):
  core = lax.axis_index("c")
  t = lax.axis_index("s")
  c0 = t * W
  iota = lax.iota(jnp.int32, 16)

  def scan_feature(fr, width):
    c0v = lax.broadcast_in_dim(c0, (16,), ())
    c1v = lax.broadcast_in_dim(c0 + width, (16,), ())

    def scan_vec(i, carry):
      nh, bvec = carry
      v = vals_v[fr, pl.ds(i * 16, 16)]
      m = (v >= c0v) & (v < c1v)
      plsc.store_compressed(hv.at[pl.ds(nh, 16)], v, mask=m)
      plsc.store_compressed(hb.at[pl.ds(nh, 16)], bvec, mask=m)
      return nh + jnp.sum(m.astype(jnp.int32)), bvec + 16

    nh, _ = lax.fori_loop(0, B // 16, scan_vec, (jnp.int32(0), iota),
                          unroll=False)
    return nh

  def unit_body(u, _):
    buf = lax.rem(u, jnp.int32(2))
    k = lax.div(u, jnp.int32(SUB))
    f = core + NC * k
    s = lax.rem(u, jnp.int32(SUB))
    fr = lax.rem(f, jnp.int32(SUB))

    @pl.when((s == 0) & (lax.rem(k, jnp.int32(4)) == 0))
    def _():
      oct_ = lax.div(f, jnp.int32(SUB))
      pltpu.sync_copy(
          values_hbm.at[pl.ds(pl.multiple_of(oct_ * SUB, SUB), SUB), :],
          vals_v)

    nh = lax.cond(
        s == 0,
        lambda: lax.cond(t == 15,
                         lambda: scan_feature(fr, W15),
                         lambda: scan_feature(fr, W)),
        lambda: smem[0])
    smem[0] = nh

    @pl.when(t < 15)
    def _():
      pltpu.sync_copy(
          tables_hbm.at[f * SUB + s, :, pl.ds(pl.multiple_of(c0, 128), W)],
          slab)

    @pl.when(t == 15)
    def _():
      pltpu.sync_copy(tables_hbm.at[f * SUB + s, :, pl.ds(15 * W, 5888)],
                      slab.at[:, pl.ds(0, 5888)])
      pltpu.sync_copy(tail_hbm.at[f * SUB + s, :, :],
                      slab.at[:, pl.ds(5888, 128)])

    @pl.when(u >= 2)
    def _():
      for o in range(2):
        @pl.when(buf == o)
        def _():
          pltpu.make_async_copy(
              outslab.at[pl.ds(0, 2048)],
              out_hbm.at[pl.ds(0, 2048)],
              wsem.at[o]).wait()
    plsc.subcore_barrier()

    nhv = lax.broadcast_in_dim(nh, (16,), ())
    obase = lax.broadcast_in_dim(buf * SLABW, (16,), ())
    ngrp = lax.div(nh + 15, jnp.int32(16))

    def scat_wait1(i, _):
      pltpu.make_async_copy(
          evbuf.at[0],
          outslab.at[oidx.at[0]],
          scsem).wait()
      return 0

    def grp_body(g, carry):
      bvec, = carry
      ring = lax.rem(g, jnp.int32(8))

      hvv = hv[pl.ds(g * 16, 16)]
      hbv = hb[pl.ds(g * 16, 16)]
      ok = bvec < nhv
      shift = jnp.where(hvv >= jnp.int32(VOCAB - 32), jnp.int32(96),
                        jnp.int32(0))
      col = lax.min(lax.max(hvv - c0v16 + shift, zeros16), jnp.int32(W - 1))
      pos0 = ((hbv >> 7) * 1024 + (hbv & 127)) + obase
      for r in range(SUB):
        gv = plsc.load_gather(slab, [jnp.full((16,), r, jnp.int32), col])
        evbuf[ring, pl.ds(r * 16, 16)] = gv
        oidx[ring, pl.ds(r * 16, 16)] = jnp.where(ok, pos0 + r * 128,
                                                  jnp.int32(JUNK))
      pltpu.async_copy(evbuf.at[ring], outslab.at[oidx.at[ring]], scsem).wait()
      return (bvec + 16,)

    c0v16 = lax.broadcast_in_dim(c0, (16,), ())
    zeros16 = jnp.zeros((16,), jnp.int32)
    lax.fori_loop(0, ngrp, grp_body, (iota,), unroll=False)

    plsc.subcore_barrier()

    for o in range(2):
      @pl.when(buf == o)
      def _():
        pltpu.async_copy(
            outslab.at[pl.ds(
                pl.multiple_of(buf * SLABW + t * 2048, 1024), 2048)],
            out_hbm.at[pl.ds(
                pl.multiple_of(((f * SUB + s) * SUB) * B + t * 2048, 1024),
                2048)],
            wsem.at[o])
    return 0

  lax.fori_loop(0, NU, unit_body, 0, unroll=False)

  for o in range(2):
    pltpu.make_async_copy(
        outslab.at[pl.ds(0, 2048)],
        out_hbm.at[pl.ds(0, 2048)],
        wsem.at[o]).wait()


@jax.jit
def _pooled_lookup(values, tables_3, tail_3):
  mesh = plsc.VectorSubcoreMesh(core_axis_name="c", subcore_axis_name="s")
  fn = pl.kernel(
      _body,
      out_type=jax.ShapeDtypeStruct((F * D * B,), jnp.float32),
      mesh=mesh,
      scratch_types=[
          pltpu.VMEM((SUB, B), jnp.int32),
          pltpu.VMEM((HCAP,), jnp.int32),
          pltpu.VMEM((HCAP,), jnp.int32),
          pltpu.VMEM((SUB, W), jnp.float32),
          pltpu.VMEM((8, 128), jnp.float32),
          pltpu.VMEM((8, 128), jnp.int32),
          pltpu.VMEM_SHARED((2 * SLABW + 128,), jnp.float32),
          pltpu.SMEM((1,), jnp.int32),
          pltpu.SemaphoreType.DMA,
          pltpu.SemaphoreType.DMA,
          pltpu.SemaphoreType.DMA((2,)),
      ],
      compiler_params=pltpu.CompilerParams(
          use_tc_tiling_on_sc=True, needs_layout_passes=False),
  )
  return fn(values, tables_3, tail_3)


def kernel(values, offsets, tables):
  del offsets
  tables_3 = jnp.transpose(tables, (0, 2, 1)).reshape(F * SUB, SUB, VOCAB)
  values_p = jnp.pad(values, ((0, 32 - F), (0, 0)))
  tail_3 = (jnp.transpose(tables[:, VOCAB - 128:, :], (0, 2, 1))
            .reshape(F * SUB, SUB, 128))
  out1 = _pooled_lookup(values_p, tables_3, tail_3)
  out = (out1.reshape(F, SUB, B // 128, SUB, 128)
         .transpose(0, 1, 3, 2, 4)
         .reshape(F, D, B)
         .transpose(2, 0, 1))
  return out
